# baseline (speedup 1.0000x reference)
import functools
import jax
import jax.numpy as jnp
from jax import lax
from jax.experimental import pallas as pl
from jax.experimental.pallas import tpu as pltpu
from jax.experimental.pallas import tpu_sc as plsc

N = 4096
DIN = 128
D = 32
M = 32
K = 32
DEPTH = 4
OUT = 6
EIN = 2 * D + 1
EH = 2 * EIN

ROWS_BLK = 512
NODE_BLK = 512


def _silu(x):
    return x * jax.nn.sigmoid(x)


def _embed_body(h_ref, we_ref, be_ref, x_ref):
    x_ref[...] = jnp.dot(h_ref[...], we_ref[...],
                         preferred_element_type=jnp.float32) + be_ref[...]


def _embed(h, We, be):
    grid = (N // ROWS_BLK,)
    return pl.pallas_call(
        _embed_body,
        grid=grid,
        in_specs=[
            pl.BlockSpec((ROWS_BLK, DIN), lambda i: (i, 0)),
            pl.BlockSpec((DIN, D), lambda i: (0, 0)),
            pl.BlockSpec((1, D), lambda i: (0, 0)),
        ],
        out_specs=pl.BlockSpec((ROWS_BLK, D), lambda i: (i, 0)),
        out_shape=jax.ShapeDtypeStruct((N, D), jnp.float32),
    )(h, We, be)


def _layer_body(x_ref, g_ref, rd_ref,
                w1_ref, b1_ref, w2_ref, b2_ref, wg_ref, bg_ref,
                wn1_ref, bn1_ref, wn2_ref, bn2_ref,
                out_ref):
    xb = x_ref[...]
    g3 = g_ref[...].reshape(NODE_BLK, K, D)
    xi3 = jnp.broadcast_to(xb[:, None, :], (NODE_BLK, K, D))
    rd3 = rd_ref[...][:, :, None]
    edge_in = jnp.concatenate([xi3, g3, rd3], axis=-1)
    e2 = edge_in.reshape(NODE_BLK * K, EIN)
    h1 = _silu(jnp.dot(e2, w1_ref[...],
                       preferred_element_type=jnp.float32) + b1_ref[...])
    m = _silu(jnp.dot(h1, w2_ref[...],
                      preferred_element_type=jnp.float32) + b2_ref[...])
    gate = jax.nn.sigmoid(jnp.dot(m, wg_ref[...],
                                  preferred_element_type=jnp.float32)
                          + bg_ref[...])
    m = m * gate
    mi = jnp.sum(m.reshape(NODE_BLK, K, M), axis=1)
    node_in = jnp.concatenate([xb, mi], axis=-1)
    n1 = _silu(jnp.dot(node_in, wn1_ref[...],
                       preferred_element_type=jnp.float32) + bn1_ref[...])
    out_ref[...] = jnp.dot(n1, wn2_ref[...],
                           preferred_element_type=jnp.float32) \
        + bn2_ref[...] + xb


def _layer(x, g, rd, w1, b1, w2, b2, wg, bg, wn1, bn1, wn2, bn2):
    grid = (N // NODE_BLK,)
    E = NODE_BLK * K
    full = lambda r, c: pl.BlockSpec((r, c), lambda i: (0, 0))
    return pl.pallas_call(
        _layer_body,
        grid=grid,
        in_specs=[
            pl.BlockSpec((NODE_BLK, D), lambda i: (i, 0)),
            pl.BlockSpec((E, D), lambda i: (i, 0)),
            pl.BlockSpec((NODE_BLK, K), lambda i: (i, 0)),
            full(EIN, EH), pl.BlockSpec((1, EH), lambda i: (0, 0)),
            full(EH, M), pl.BlockSpec((1, M), lambda i: (0, 0)),
            full(M, 1), pl.BlockSpec((1, 1), lambda i: (0, 0)),
            full(2 * D, 2 * D), pl.BlockSpec((1, 2 * D), lambda i: (0, 0)),
            full(2 * D, D), pl.BlockSpec((1, D), lambda i: (0, 0)),
        ],
        out_specs=pl.BlockSpec((NODE_BLK, D), lambda i: (i, 0)),
        out_shape=jax.ShapeDtypeStruct((N, D), jnp.float32),
    )(x, g, rd, w1, b1, w2, b2, wg, bg, wn1, bn1, wn2, bn2)


NW = 32
GCHUNK = 1024
GNCH = (N * K) // NW // GCHUNK


def _sc_gather_body(idx_hbm, table_hbm, out_hbm,
                    idx_v0, idx_v1, rows_v0, rows_v1, sem0, sem1):
    wid = lax.axis_index("s") * 2 + lax.axis_index("c")
    base = wid * (GNCH * GCHUNK)
    idx_bufs = (idx_v0, idx_v1)
    row_bufs = (rows_v0, rows_v1)
    sems = (sem0, sem1)
    copies = [None, None]
    pltpu.sync_copy(idx_hbm.at[pl.ds(base, GCHUNK)], idx_v0)
    copies[0] = pltpu.async_copy(table_hbm.at[idx_v0], rows_v0, sem0)
    for c in range(GNCH):
        b = c % 2
        nb = (c + 1) % 2
        if c + 1 < GNCH:
            off = base + (c + 1) * GCHUNK
            pltpu.sync_copy(idx_hbm.at[pl.ds(off, GCHUNK)], idx_bufs[nb])
            copies[nb] = pltpu.async_copy(
                table_hbm.at[idx_bufs[nb]], row_bufs[nb], sems[nb])
        copies[b].wait()
        pltpu.sync_copy(row_bufs[b],
                        out_hbm.at[pl.ds(base + c * GCHUNK, GCHUNK)])


def _sc_gather(idx_flat, table):
    mesh = plsc.VectorSubcoreMesh(core_axis_name="c", subcore_axis_name="s")
    f = functools.partial(
        pl.kernel, mesh=mesh,
        compiler_params=pltpu.CompilerParams(use_tc_tiling_on_sc=False),
        out_type=jax.ShapeDtypeStruct((N * K, D), jnp.float32),
        scratch_types=[
            pltpu.VMEM((GCHUNK,), jnp.int32),
            pltpu.VMEM((GCHUNK,), jnp.int32),
            pltpu.VMEM((GCHUNK, D), jnp.float32),
            pltpu.VMEM((GCHUNK, D), jnp.float32),
            pltpu.SemaphoreType.DMA,
            pltpu.SemaphoreType.DMA,
        ],
    )(_sc_gather_body)
    return f(idx_flat, table)


TROWS = N // NW
TCH = 4
TNCH = TROWS // TCH
CANDPAD = N + 16
_F32MAX = 3.4028235e38


def _sc_topk_row(rbuf, rr, cand_val, cand_idx, oval_st, oidx_st, lr):
    lane = lax.iota(jnp.int32, 16)
    inf16 = jnp.full((16,), _F32MAX, jnp.float32)

    def p1(i, carry):
        m1, m2 = carry
        for u in range(4):
            v = rbuf[rr, pl.ds((i * 4 + u) * 16, 16)]
            hi = jnp.maximum(m1, v)
            m1 = jnp.minimum(m1, v)
            m2 = jnp.minimum(m2, hi)
        return m1, m2

    m1, m2 = lax.fori_loop(0, N // 64, p1, (inf16, inf16))
    T = jnp.max(m2)

    def p2(i, cnt):
        for u in range(4):
            j = i * 4 + u
            ivec = j * 16 + lane
            v = rbuf[rr, pl.ds(j * 16, 16)]
            mask = v <= T
            cum = plsc.cumsum(mask.astype(jnp.int32))
            pos = cnt + cum - 1
            plsc.store_scatter(cand_val, [pos], v, mask=mask)
            plsc.store_scatter(cand_idx, [pos], ivec, mask=mask)
            cnt = cnt + jnp.max(cum)
        return cnt

    cnt = lax.fori_loop(0, N // 64, p2, jnp.int32(0))

    def p3(c, carry):
        blk, bli, bhk, bhi = carry
        pos = c * 16 + lane
        ck = cand_val[pl.ds(c * 16, 16)]
        ci = cand_idx[pl.ds(c * 16, 16)]
        ck = jnp.where(pos < cnt, ck, inf16)
        ck, ci = plsc.sort_key_val(ck, ci)
        rk, ri = jnp.flip(ck, 0), jnp.flip(ci, 0)
        m = bhk <= rk
        tk = jnp.where(m, bhk, rk)
        ti = jnp.where(m, bhi, ri)
        tk, ti = plsc.sort_key_val(tk, ti)
        rtk, rti = jnp.flip(tk, 0), jnp.flip(ti, 0)
        m = blk <= rtk
        nlk = jnp.where(m, blk, rtk)
        nli = jnp.where(m, bli, rti)
        nhk = jnp.where(m, rtk, blk)
        nhi = jnp.where(m, rti, bli)
        blk, bli = plsc.sort_key_val(nlk, nli)
        bhk, bhi = plsc.sort_key_val(nhk, nhi)
        return blk, bli, bhk, bhi

    zero16 = jnp.zeros((16,), jnp.int32)
    nch = (cnt + 15) // 16
    blk, bli, bhk, bhi = lax.fori_loop(
        0, nch, p3, (inf16, zero16, inf16, zero16))

    oval_st[lr, pl.ds(0, 16)] = blk
    oval_st[lr, pl.ds(16, 16)] = bhk
    oidx_st[lr, pl.ds(0, 16)] = bli
    oidx_st[lr, pl.ds(16, 16)] = bhi


def _sc_topk_body(dist_hbm, oval_hbm, oidx_hbm,
                  rb0, rb1, cand_val, cand_idx, oval_st, oidx_st, sem0, sem1):
    wid = lax.axis_index("s") * 2 + lax.axis_index("c")
    base = wid * TROWS
    pltpu.async_copy(dist_hbm.at[pl.ds(base, TCH)], rb0, sem0)
    pltpu.async_copy(dist_hbm.at[pl.ds(base + TCH, TCH)], rb1, sem1)

    def half(rbuf, sem, c):
        pltpu.make_async_copy(dist_hbm.at[pl.ds(base, TCH)], rbuf, sem).wait()

        def rowb(r, z):
            _sc_topk_row(rbuf, r, cand_val, cand_idx,
                         oval_st, oidx_st, c * TCH + r)
            return z

        lax.fori_loop(0, TCH, rowb, jnp.int32(0))
        nxt = jnp.minimum(base + (c + 2) * TCH, N - TCH)
        pltpu.async_copy(dist_hbm.at[pl.ds(nxt, TCH)], rbuf, sem)

    def pair(p, z):
        half(rb0, sem0, 2 * p)
        half(rb1, sem1, 2 * p + 1)
        return z

    lax.fori_loop(0, TNCH // 2, pair, jnp.int32(0))
    pltpu.make_async_copy(dist_hbm.at[pl.ds(base, TCH)], rb0, sem0).wait()
    pltpu.make_async_copy(dist_hbm.at[pl.ds(base, TCH)], rb1, sem1).wait()
    pltpu.sync_copy(oval_st, oval_hbm.at[pl.ds(base, TROWS)])
    pltpu.sync_copy(oidx_st, oidx_hbm.at[pl.ds(base, TROWS)])


def _sc_topk(dist):
    mesh = plsc.VectorSubcoreMesh(core_axis_name="c", subcore_axis_name="s")
    f = functools.partial(
        pl.kernel, mesh=mesh,
        compiler_params=pltpu.CompilerParams(use_tc_tiling_on_sc=False,
                                             needs_layout_passes=False),
        out_type=(jax.ShapeDtypeStruct((N, K), jnp.float32),
                  jax.ShapeDtypeStruct((N, K), jnp.int32)),
        scratch_types=[
            pltpu.VMEM((TCH, N), jnp.float32),
            pltpu.VMEM((TCH, N), jnp.float32),
            pltpu.VMEM((CANDPAD,), jnp.float32),
            pltpu.VMEM((CANDPAD,), jnp.int32),
            pltpu.VMEM((TROWS, K), jnp.float32),
            pltpu.VMEM((TROWS, K), jnp.int32),
            pltpu.SemaphoreType.DMA,
            pltpu.SemaphoreType.DMA,
        ],
    )(_sc_topk_body)
    return f(dist)


def _head_body(x_ref, wh1_ref, bh1_ref, wh2_ref, bh2_ref, out_ref):
    t = jax.nn.relu(jnp.dot(x_ref[...], wh1_ref[...],
                            preferred_element_type=jnp.float32)
                    + bh1_ref[...])
    out_ref[...] = jnp.dot(t, wh2_ref[...],
                           preferred_element_type=jnp.float32) + bh2_ref[...]


def _head(x, Wh1, bh1, Wh2, bh2):
    grid = (N // ROWS_BLK,)
    return pl.pallas_call(
        _head_body,
        grid=grid,
        in_specs=[
            pl.BlockSpec((ROWS_BLK, D), lambda i: (i, 0)),
            pl.BlockSpec((D, D), lambda i: (0, 0)),
            pl.BlockSpec((1, D), lambda i: (0, 0)),
            pl.BlockSpec((D, OUT), lambda i: (0, 0)),
            pl.BlockSpec((1, OUT), lambda i: (0, 0)),
        ],
        out_specs=pl.BlockSpec((ROWS_BLK, OUT), lambda i: (i, 0)),
        out_shape=jax.ShapeDtypeStruct((N, OUT), jnp.float32),
    )(x, Wh1, bh1, Wh2, bh2)


def kernel(h, pos, We, be, W1, b1, W2, b2, Wg, bg, Wn1, bn1, Wn2, bn2,
           Wh1, bh1, Wh2, bh2):
    h0 = h[0]
    x = _embed(h0, We, be.reshape(1, D))

    sq = jnp.sum(pos * pos, axis=-1)
    dist = (sq[:, :, None] + sq[:, None, :]
            - 2.0 * jnp.einsum('bid,bjd->bij', pos, pos))[0]

    rel_dist, idx = _sc_topk(dist)

    idx_flat = idx.reshape(-1)
    for l in range(DEPTH):
        g = _sc_gather(idx_flat, x)
        x = _layer(x, g, rel_dist,
                   W1[l], b1[l].reshape(1, EH),
                   W2[l], b2[l].reshape(1, M),
                   Wg[l], bg[l].reshape(1, 1),
                   Wn1[l], bn1[l].reshape(1, 2 * D),
                   Wn2[l], bn2[l].reshape(1, D))

    y = _head(x, Wh1, bh1.reshape(1, D), Wh2, bh2.reshape(1, OUT))
    return y[None]


# speedup vs baseline: 1.0138x; 1.0138x over previous
import functools
import jax
import jax.numpy as jnp
from jax import lax
from jax.experimental import pallas as pl
from jax.experimental.pallas import tpu as pltpu
from jax.experimental.pallas import tpu_sc as plsc

N = 4096
DIN = 128
D = 32
M = 32
K = 32
DEPTH = 4
OUT = 6
EIN = 2 * D + 1
EH = 2 * EIN

ROWS_BLK = 512
NODE_BLK = 128


def _silu(x):
    return x * jax.nn.sigmoid(x)


def _embed_body(h_ref, we_ref, be_ref, x_ref):
    x_ref[...] = jnp.dot(h_ref[...], we_ref[...],
                         preferred_element_type=jnp.float32) + be_ref[...]


def _embed(h, We, be):
    grid = (N // ROWS_BLK,)
    return pl.pallas_call(
        _embed_body,
        grid=grid,
        in_specs=[
            pl.BlockSpec((ROWS_BLK, DIN), lambda i: (i, 0)),
            pl.BlockSpec((DIN, D), lambda i: (0, 0)),
            pl.BlockSpec((1, D), lambda i: (0, 0)),
        ],
        out_specs=pl.BlockSpec((ROWS_BLK, D), lambda i: (i, 0)),
        out_shape=jax.ShapeDtypeStruct((N, D), jnp.float32),
    )(h, We, be)


def _layer_body(x_ref, g_ref, rd_ref,
                w1_ref, b1_ref, w2_ref, b2_ref, wg_ref, bg_ref,
                wn1_ref, bn1_ref, wn2_ref, bn2_ref,
                out_ref):
    xb = x_ref[...]
    g3 = g_ref[...].reshape(NODE_BLK, K, D)
    xi3 = jnp.broadcast_to(xb[:, None, :], (NODE_BLK, K, D))
    rd3 = rd_ref[...][:, :, None]
    edge_in = jnp.concatenate([xi3, g3, rd3], axis=-1)
    e2 = edge_in.reshape(NODE_BLK * K, EIN)
    h1 = _silu(jnp.dot(e2, w1_ref[...],
                       preferred_element_type=jnp.float32) + b1_ref[...])
    m = _silu(jnp.dot(h1, w2_ref[...],
                      preferred_element_type=jnp.float32) + b2_ref[...])
    gate = jax.nn.sigmoid(jnp.dot(m, wg_ref[...],
                                  preferred_element_type=jnp.float32)
                          + bg_ref[...])
    m = m * gate
    mi = jnp.sum(m.reshape(NODE_BLK, K, M), axis=1)
    node_in = jnp.concatenate([xb, mi], axis=-1)
    n1 = _silu(jnp.dot(node_in, wn1_ref[...],
                       preferred_element_type=jnp.float32) + bn1_ref[...])
    out_ref[...] = jnp.dot(n1, wn2_ref[...],
                           preferred_element_type=jnp.float32) \
        + bn2_ref[...] + xb


def _layer(x, g, rd, w1, b1, w2, b2, wg, bg, wn1, bn1, wn2, bn2):
    grid = (N // NODE_BLK,)
    E = NODE_BLK * K
    full = lambda r, c: pl.BlockSpec((r, c), lambda i: (0, 0))
    return pl.pallas_call(
        _layer_body,
        grid=grid,
        in_specs=[
            pl.BlockSpec((NODE_BLK, D), lambda i: (i, 0)),
            pl.BlockSpec((E, D), lambda i: (i, 0)),
            pl.BlockSpec((NODE_BLK, K), lambda i: (i, 0)),
            full(EIN, EH), pl.BlockSpec((1, EH), lambda i: (0, 0)),
            full(EH, M), pl.BlockSpec((1, M), lambda i: (0, 0)),
            full(M, 1), pl.BlockSpec((1, 1), lambda i: (0, 0)),
            full(2 * D, 2 * D), pl.BlockSpec((1, 2 * D), lambda i: (0, 0)),
            full(2 * D, D), pl.BlockSpec((1, D), lambda i: (0, 0)),
        ],
        out_specs=pl.BlockSpec((NODE_BLK, D), lambda i: (i, 0)),
        out_shape=jax.ShapeDtypeStruct((N, D), jnp.float32),
    )(x, g, rd, w1, b1, w2, b2, wg, bg, wn1, bn1, wn2, bn2)


NW = 32
GCHUNK = 1024
GNCH = (N * K) // NW // GCHUNK


def _sc_gather_body(idx_hbm, table_hbm, out_hbm,
                    idx_v0, idx_v1, rows_v0, rows_v1, sem0, sem1):
    wid = lax.axis_index("s") * 2 + lax.axis_index("c")
    base = wid * (GNCH * GCHUNK)
    idx_bufs = (idx_v0, idx_v1)
    row_bufs = (rows_v0, rows_v1)
    sems = (sem0, sem1)
    copies = [None, None]
    pltpu.sync_copy(idx_hbm.at[pl.ds(base, GCHUNK)], idx_v0)
    copies[0] = pltpu.async_copy(table_hbm.at[idx_v0], rows_v0, sem0)
    for c in range(GNCH):
        b = c % 2
        nb = (c + 1) % 2
        if c + 1 < GNCH:
            off = base + (c + 1) * GCHUNK
            pltpu.sync_copy(idx_hbm.at[pl.ds(off, GCHUNK)], idx_bufs[nb])
            copies[nb] = pltpu.async_copy(
                table_hbm.at[idx_bufs[nb]], row_bufs[nb], sems[nb])
        copies[b].wait()
        pltpu.sync_copy(row_bufs[b],
                        out_hbm.at[pl.ds(base + c * GCHUNK, GCHUNK)])


def _sc_gather(idx_flat, table):
    mesh = plsc.VectorSubcoreMesh(core_axis_name="c", subcore_axis_name="s")
    f = functools.partial(
        pl.kernel, mesh=mesh,
        compiler_params=pltpu.CompilerParams(use_tc_tiling_on_sc=False),
        out_type=jax.ShapeDtypeStruct((N * K, D), jnp.float32),
        scratch_types=[
            pltpu.VMEM((GCHUNK,), jnp.int32),
            pltpu.VMEM((GCHUNK,), jnp.int32),
            pltpu.VMEM((GCHUNK, D), jnp.float32),
            pltpu.VMEM((GCHUNK, D), jnp.float32),
            pltpu.SemaphoreType.DMA,
            pltpu.SemaphoreType.DMA,
        ],
    )(_sc_gather_body)
    return f(idx_flat, table)


TROWS = N // NW
TCH = 4
TNCH = TROWS // TCH
CANDPAD = N + 16
_F32MAX = 3.4028235e38


def _sc_topk_row(rbuf, rr, cand_val, cand_idx, oval_st, oidx_st, lr):
    lane = lax.iota(jnp.int32, 16)
    inf16 = jnp.full((16,), _F32MAX, jnp.float32)

    def p1(i, carry):
        m1, m2 = carry
        for u in range(4):
            v = rbuf[rr, pl.ds((i * 4 + u) * 16, 16)]
            hi = jnp.maximum(m1, v)
            m1 = jnp.minimum(m1, v)
            m2 = jnp.minimum(m2, hi)
        return m1, m2

    m1, m2 = lax.fori_loop(0, N // 64, p1, (inf16, inf16))
    T = jnp.max(m2)

    def p2(i, cnt):
        for u in range(4):
            j = i * 4 + u
            ivec = j * 16 + lane
            v = rbuf[rr, pl.ds(j * 16, 16)]
            mask = v <= T
            cum = plsc.cumsum(mask.astype(jnp.int32))
            pos = cnt + cum - 1
            plsc.store_scatter(cand_val, [pos], v, mask=mask)
            plsc.store_scatter(cand_idx, [pos], ivec, mask=mask)
            cnt = cnt + jnp.max(cum)
        return cnt

    cnt = lax.fori_loop(0, N // 64, p2, jnp.int32(0))

    def p3(c, carry):
        blk, bli, bhk, bhi = carry
        pos = c * 16 + lane
        ck = cand_val[pl.ds(c * 16, 16)]
        ci = cand_idx[pl.ds(c * 16, 16)]
        ck = jnp.where(pos < cnt, ck, inf16)
        ck, ci = plsc.sort_key_val(ck, ci)
        rk, ri = jnp.flip(ck, 0), jnp.flip(ci, 0)
        m = bhk <= rk
        tk = jnp.where(m, bhk, rk)
        ti = jnp.where(m, bhi, ri)
        tk, ti = plsc.sort_key_val(tk, ti)
        rtk, rti = jnp.flip(tk, 0), jnp.flip(ti, 0)
        m = blk <= rtk
        nlk = jnp.where(m, blk, rtk)
        nli = jnp.where(m, bli, rti)
        nhk = jnp.where(m, rtk, blk)
        nhi = jnp.where(m, rti, bli)
        blk, bli = plsc.sort_key_val(nlk, nli)
        bhk, bhi = plsc.sort_key_val(nhk, nhi)
        return blk, bli, bhk, bhi

    zero16 = jnp.zeros((16,), jnp.int32)
    nch = (cnt + 15) // 16
    blk, bli, bhk, bhi = lax.fori_loop(
        0, nch, p3, (inf16, zero16, inf16, zero16))

    oval_st[lr, pl.ds(0, 16)] = blk
    oval_st[lr, pl.ds(16, 16)] = bhk
    oidx_st[lr, pl.ds(0, 16)] = bli
    oidx_st[lr, pl.ds(16, 16)] = bhi


def _sc_topk_body(dist_hbm, oval_hbm, oidx_hbm,
                  rb0, rb1, cand_val, cand_idx, oval_st, oidx_st, sem0, sem1):
    wid = lax.axis_index("s") * 2 + lax.axis_index("c")
    base = wid * TROWS
    pltpu.async_copy(dist_hbm.at[pl.ds(base, TCH)], rb0, sem0)
    pltpu.async_copy(dist_hbm.at[pl.ds(base + TCH, TCH)], rb1, sem1)

    def half(rbuf, sem, c):
        pltpu.make_async_copy(dist_hbm.at[pl.ds(base, TCH)], rbuf, sem).wait()

        def rowb(r, z):
            _sc_topk_row(rbuf, r, cand_val, cand_idx,
                         oval_st, oidx_st, c * TCH + r)
            return z

        lax.fori_loop(0, TCH, rowb, jnp.int32(0))
        nxt = jnp.minimum(base + (c + 2) * TCH, N - TCH)
        pltpu.async_copy(dist_hbm.at[pl.ds(nxt, TCH)], rbuf, sem)

    def pair(p, z):
        half(rb0, sem0, 2 * p)
        half(rb1, sem1, 2 * p + 1)
        return z

    lax.fori_loop(0, TNCH // 2, pair, jnp.int32(0))
    pltpu.make_async_copy(dist_hbm.at[pl.ds(base, TCH)], rb0, sem0).wait()
    pltpu.make_async_copy(dist_hbm.at[pl.ds(base, TCH)], rb1, sem1).wait()
    pltpu.sync_copy(oval_st, oval_hbm.at[pl.ds(base, TROWS)])
    pltpu.sync_copy(oidx_st, oidx_hbm.at[pl.ds(base, TROWS)])


def _sc_topk(dist):
    mesh = plsc.VectorSubcoreMesh(core_axis_name="c", subcore_axis_name="s")
    f = functools.partial(
        pl.kernel, mesh=mesh,
        compiler_params=pltpu.CompilerParams(use_tc_tiling_on_sc=False,
                                             needs_layout_passes=False),
        out_type=(jax.ShapeDtypeStruct((N, K), jnp.float32),
                  jax.ShapeDtypeStruct((N, K), jnp.int32)),
        scratch_types=[
            pltpu.VMEM((TCH, N), jnp.float32),
            pltpu.VMEM((TCH, N), jnp.float32),
            pltpu.VMEM((CANDPAD,), jnp.float32),
            pltpu.VMEM((CANDPAD,), jnp.int32),
            pltpu.VMEM((TROWS, K), jnp.float32),
            pltpu.VMEM((TROWS, K), jnp.int32),
            pltpu.SemaphoreType.DMA,
            pltpu.SemaphoreType.DMA,
        ],
    )(_sc_topk_body)
    return f(dist)


def _head_body(x_ref, wh1_ref, bh1_ref, wh2_ref, bh2_ref, out_ref):
    t = jax.nn.relu(jnp.dot(x_ref[...], wh1_ref[...],
                            preferred_element_type=jnp.float32)
                    + bh1_ref[...])
    out_ref[...] = jnp.dot(t, wh2_ref[...],
                           preferred_element_type=jnp.float32) + bh2_ref[...]


def _head(x, Wh1, bh1, Wh2, bh2):
    grid = (N // ROWS_BLK,)
    return pl.pallas_call(
        _head_body,
        grid=grid,
        in_specs=[
            pl.BlockSpec((ROWS_BLK, D), lambda i: (i, 0)),
            pl.BlockSpec((D, D), lambda i: (0, 0)),
            pl.BlockSpec((1, D), lambda i: (0, 0)),
            pl.BlockSpec((D, OUT), lambda i: (0, 0)),
            pl.BlockSpec((1, OUT), lambda i: (0, 0)),
        ],
        out_specs=pl.BlockSpec((ROWS_BLK, OUT), lambda i: (i, 0)),
        out_shape=jax.ShapeDtypeStruct((N, OUT), jnp.float32),
    )(x, Wh1, bh1, Wh2, bh2)


def kernel(h, pos, We, be, W1, b1, W2, b2, Wg, bg, Wn1, bn1, Wn2, bn2,
           Wh1, bh1, Wh2, bh2):
    h0 = h[0]
    x = _embed(h0, We, be.reshape(1, D))

    sq = jnp.sum(pos * pos, axis=-1)
    dist = (sq[:, :, None] + sq[:, None, :]
            - 2.0 * jnp.einsum('bid,bjd->bij', pos, pos))[0]

    rel_dist, idx = _sc_topk(dist)

    idx_flat = idx.reshape(-1)
    for l in range(DEPTH):
        g = _sc_gather(idx_flat, x)
        x = _layer(x, g, rel_dist,
                   W1[l], b1[l].reshape(1, EH),
                   W2[l], b2[l].reshape(1, M),
                   Wg[l], bg[l].reshape(1, 1),
                   Wn1[l], bn1[l].reshape(1, 2 * D),
                   Wn2[l], bn2[l].reshape(1, D))

    y = _head(x, Wh1, bh1.reshape(1, D), Wh2, bh2.reshape(1, OUT))
    return y[None]
